# baseline (speedup 1.0000x reference)
import functools
import jax
import jax.numpy as jnp
from jax import lax
from jax.experimental import pallas as pl
from jax.experimental.pallas import tpu as pltpu
from jax.experimental.pallas import tpu_sc as plsc

EMBED_DIM = 64
PAIR_DIM = 2 * EMBED_DIM
NUM_CORES = 2
NUM_SUBCORES = 16
NUM_WORKERS = NUM_CORES * NUM_SUBCORES
CHUNK = 512
SEL_BLOCK = 1024


def _sc_gather(table2, idx2, num_indices):
    mesh = plsc.VectorSubcoreMesh(core_axis_name="c", subcore_axis_name="s")
    per_worker = num_indices // NUM_WORKERS
    num_chunks = per_worker // CHUNK

    @functools.partial(
        pl.kernel,
        mesh=mesh,
        out_type=jax.ShapeDtypeStruct((num_indices, PAIR_DIM), jnp.float32),
        scratch_types=[
            pltpu.VMEM((CHUNK,), jnp.int32),
            pltpu.VMEM((CHUNK, PAIR_DIM), jnp.float32),
            pltpu.SemaphoreType.DMA,
        ],
    )
    def sc_kernel(table_hbm, idx_hbm, out_hbm, idx_v, rows_v, sem):
        wid = lax.axis_index("s") * NUM_CORES + lax.axis_index("c")
        base = wid * per_worker

        @pl.loop(0, num_chunks)
        def _(c):
            start = base + c * CHUNK
            pltpu.sync_copy(idx_hbm.at[pl.ds(start, CHUNK)], idx_v)
            pltpu.async_copy(table_hbm.at[idx_v], rows_v, sem).wait()
            pltpu.sync_copy(rows_v, out_hbm.at[pl.ds(start, CHUNK)])

    return sc_kernel(table2, idx2)


def _tc_select(pairs, pmask, num_indices):
    def sel_kernel(pairs_ref, pm_ref, out_ref):
        pm = pm_ref[...]
        out_ref[...] = (pairs_ref[:, :EMBED_DIM] * (1.0 - pm)
                        + pairs_ref[:, EMBED_DIM:] * pm)

    grid = (num_indices // SEL_BLOCK,)
    return pl.pallas_call(
        sel_kernel,
        grid=grid,
        in_specs=[
            pl.BlockSpec((SEL_BLOCK, PAIR_DIM), lambda i: (i, 0)),
            pl.BlockSpec((SEL_BLOCK, 1), lambda i: (i, 0)),
        ],
        out_specs=pl.BlockSpec((SEL_BLOCK, EMBED_DIM), lambda i: (i, 0)),
        out_shape=jax.ShapeDtypeStruct((num_indices, EMBED_DIM),
                                       jnp.float32),
    )(pairs, pmask)


def kernel(batchinput, weight):
    batch, seq = batchinput.shape
    num_indices = batch * seq
    idx_flat = batchinput.reshape(num_indices)
    idx2 = lax.shift_right_logical(idx_flat, 1)
    pmask = (idx_flat & 1).astype(jnp.float32).reshape(num_indices, 1)
    table2 = weight.reshape(weight.shape[0] // 2, PAIR_DIM)

    pairs = _sc_gather(table2, idx2, num_indices)
    out = _tc_select(pairs, pmask, num_indices)
    return out.reshape(batch, seq, EMBED_DIM)


# speedup vs baseline: 1.2551x; 1.2551x over previous
import functools
import jax
import jax.numpy as jnp
from jax import lax
from jax.experimental import pallas as pl
from jax.experimental.pallas import tpu as pltpu
from jax.experimental.pallas import tpu_sc as plsc

EMBED_DIM = 64
PAIR_DIM = 2 * EMBED_DIM
NUM_CORES = 2
NUM_SUBCORES = 16
NUM_WORKERS = NUM_CORES * NUM_SUBCORES
CHUNK = 512
SEL_ROWS = 16


def _sc_gather(table2, idx2, num_indices):
    mesh = plsc.VectorSubcoreMesh(core_axis_name="c", subcore_axis_name="s")
    per_worker = num_indices // NUM_WORKERS
    num_chunks = per_worker // CHUNK

    @functools.partial(
        pl.kernel,
        mesh=mesh,
        out_type=jax.ShapeDtypeStruct((num_indices, PAIR_DIM), jnp.float32),
        scratch_types=[
            pltpu.VMEM((CHUNK,), jnp.int32),
            pltpu.VMEM((CHUNK, PAIR_DIM), jnp.float32),
            pltpu.SemaphoreType.DMA,
        ],
    )
    def sc_kernel(table_hbm, idx_hbm, out_hbm, idx_v, rows_v, sem):
        wid = lax.axis_index("s") * NUM_CORES + lax.axis_index("c")
        base = wid * per_worker

        @pl.loop(0, num_chunks)
        def _(c):
            start = base + c * CHUNK
            pltpu.sync_copy(idx_hbm.at[pl.ds(start, CHUNK)], idx_v)
            pltpu.async_copy(table_hbm.at[idx_v], rows_v, sem).wait()
            pltpu.sync_copy(rows_v, out_hbm.at[pl.ds(start, CHUNK)])

    return sc_kernel(table2, idx2)


def _tc_select(pairs, idx_lanes, batch, seq):
    rows_per_step = SEL_ROWS * seq

    lane_rows = rows_per_step // 128

    def sel_kernel(pairs_ref, idx_ref, out_ref):
        pm = (idx_ref[...] & 1).astype(jnp.float32)
        pm3 = pm.reshape(lane_rows, 128, 1)
        pairs3 = pairs_ref[...].reshape(lane_rows, 128, PAIR_DIM)
        sel = (pairs3[:, :, :EMBED_DIM] * (1.0 - pm3)
               + pairs3[:, :, EMBED_DIM:] * pm3)
        out_ref[...] = sel.reshape(SEL_ROWS, seq, EMBED_DIM)

    grid = (batch // SEL_ROWS,)
    return pl.pallas_call(
        sel_kernel,
        grid=grid,
        in_specs=[
            pl.BlockSpec((rows_per_step, PAIR_DIM), lambda i: (i, 0)),
            pl.BlockSpec((1, rows_per_step // 128, 128),
                         lambda i: (i, 0, 0)),
        ],
        out_specs=pl.BlockSpec((SEL_ROWS, seq, EMBED_DIM),
                               lambda i: (i, 0, 0)),
        out_shape=jax.ShapeDtypeStruct((batch, seq, EMBED_DIM),
                                       jnp.float32),
    )(pairs, idx_lanes)


def kernel(batchinput, weight):
    batch, seq = batchinput.shape
    num_indices = batch * seq
    idx_flat = batchinput.reshape(num_indices)
    idx2 = lax.shift_right_logical(idx_flat, 1)
    rows_per_step = SEL_ROWS * seq
    idx_lanes = batchinput.reshape(num_indices // rows_per_step,
                                   rows_per_step // 128, 128)
    table2 = weight.reshape(weight.shape[0] // 2, PAIR_DIM)

    pairs = _sc_gather(table2, idx2, num_indices)
    return _tc_select(pairs, idx_lanes, batch, seq)
